# baseline (speedup 1.0000x reference)
import functools

import jax
import jax.numpy as jnp
from jax import lax
from jax.experimental import pallas as pl
from jax.experimental.pallas import tpu as pltpu
from jax.experimental.pallas import tpu_sc as plsc

B = 16384
HASH_POWER = 8
NUM_BINS = 256
FEAT_DIM = 32
FPAD = 128
BLK = 2048


def _tc_hash_body(st_ref, at_ref, bt_ref, rt_ref, proj_ref, lut_ref,
                  scale_ref, bias_ref, hash_ref, tabn_ref, pT_ref):
    i = pl.program_id(0)

    @pl.when(i == 0)
    def _():
        pT_ref[...] = jnp.transpose(proj_ref[...])
        tab = lut_ref[...]
        mu = tab.mean(axis=1, keepdims=True)
        var = ((tab - mu) ** 2).mean(axis=1, keepdims=True)
        normed = (tab - mu) * lax.rsqrt(var + 1e-6)
        normed = (normed * jnp.reshape(scale_ref[...], (1, FEAT_DIM))
                  + jnp.reshape(bias_ref[...], (1, FEAT_DIM)))
        tabn_ref[...] = jnp.concatenate(
            [normed, jnp.zeros((NUM_BINS, FPAD - FEAT_DIM), jnp.float32)],
            axis=1)

    def kdot(xgT, lo, hi):
        return jnp.dot(jnp.transpose(xgT), pT_ref[pl.ds(lo, hi - lo), :],
                       preferred_element_type=jnp.float32)
    ysr = kdot(st_ref[...], 0, 64)
    ysr = ysr + kdot(at_ref[...], 64, 320)
    ysr = ysr + kdot(bt_ref[...], 320, 832)
    ysr = ysr + kdot(rt_ref[...], 832, 960)
    ys = jnp.transpose(ysr)
    powers = jnp.int32(1) << lax.broadcasted_iota(
        jnp.int32, (HASH_POWER, 1), 0)
    bits = jnp.where(ys > 0, powers, 0)
    hash_ref[...] = bits.sum(axis=0)


def _tc_hash(st, at, bt, rt, proj, lut, scale, bias):
    grid = B // BLK
    return pl.pallas_call(
        _tc_hash_body,
        grid=(grid,),
        in_specs=[
            pl.BlockSpec((64, BLK), lambda i: (0, i)),
            pl.BlockSpec((256, BLK), lambda i: (0, i)),
            pl.BlockSpec((512, BLK), lambda i: (0, i)),
            pl.BlockSpec((128, BLK), lambda i: (0, i)),
            pl.BlockSpec((HASH_POWER, 960), lambda i: (0, 0)),
            pl.BlockSpec((NUM_BINS, FEAT_DIM), lambda i: (0, 0)),
            pl.BlockSpec((FEAT_DIM,), lambda i: (0,)),
            pl.BlockSpec((FEAT_DIM,), lambda i: (0,)),
        ],
        scratch_shapes=[pltpu.VMEM((960, HASH_POWER), jnp.float32)],
        out_specs=[
            pl.BlockSpec((BLK,), lambda i: (i,)),
            pl.BlockSpec((NUM_BINS, FPAD), lambda i: (0, 0)),
        ],
        out_shape=[
            jax.ShapeDtypeStruct((B,), jnp.int32),
            jax.ShapeDtypeStruct((NUM_BINS, FPAD), jnp.float32),
        ],
    )(st, at, bt, rt, proj, lut, scale, bias)


def _make_sc_gather():
    info = plsc.get_sparse_core_info()
    nw = info.num_cores * info.num_subcores
    bpw = B // nw
    chunk = 128
    nchunk = bpw // chunk
    mesh = plsc.VectorSubcoreMesh(core_axis_name="c", subcore_axis_name="s")

    @functools.partial(
        pl.kernel, mesh=mesh,
        out_type=jax.ShapeDtypeStruct((B, FPAD), jnp.float32),
        scratch_types=[
            pltpu.VMEM((bpw,), jnp.int32),
            pltpu.VMEM((bpw, FPAD), jnp.float32),
            pltpu.SemaphoreType.DMA,
            pltpu.SemaphoreType.DMA,
        ],
    )
    def sc_gather(table_hbm, idx_hbm, out_hbm, idx_v, rows_v, gsem, wsem):
        wid = lax.axis_index("s") * info.num_cores + lax.axis_index("c")
        base = wid * bpw
        pltpu.sync_copy(idx_hbm.at[pl.ds(base, bpw)], idx_v)
        gathers = [
            pltpu.async_copy(
                table_hbm.at[idx_v.at[pl.ds(j * chunk, chunk)]],
                rows_v.at[pl.ds(j * chunk, chunk)], gsem)
            for j in range(nchunk)
        ]
        writes = []
        for j in range(nchunk):
            gathers[j].wait()
            writes.append(pltpu.async_copy(
                rows_v.at[pl.ds(j * chunk, chunk)],
                out_hbm.at[pl.ds(base + j * chunk, chunk)], wsem))
        for w in writes:
            w.wait()

    return sc_gather, nw, nchunk, chunk


def kernel(self_ob, agents_ob, boxes_ob, ramps_ob, proj_mat, lookup,
           ln_scale, ln_bias, train):
    st = self_ob.T
    at = agents_ob.reshape(B, 256).T
    bt = boxes_ob.reshape(B, 512).T
    rt = ramps_ob.reshape(B, 128).T
    hash_val, table_n = _tc_hash(st, at, bt, rt, proj_mat, lookup,
                                 ln_scale, ln_bias)

    sc_gather, nw, nchunk, chunk = _make_sc_gather()
    out_p = sc_gather(table_n, hash_val)
    return out_p[:, :FEAT_DIM]


# speedup vs baseline: 1.0021x; 1.0021x over previous
import functools

import jax
import jax.numpy as jnp
from jax import lax
from jax.experimental import pallas as pl
from jax.experimental.pallas import tpu as pltpu
from jax.experimental.pallas import tpu_sc as plsc

B = 16384
HASH_POWER = 8
NUM_BINS = 256
FEAT_DIM = 32
FPAD = 128
BLK = 2048


def _tc_hash_body(st_ref, at_ref, bt_ref, rt_ref, proj_ref, lut_ref,
                  scale_ref, bias_ref, hash_ref, tabn_ref, pT_ref):
    i = pl.program_id(0)

    @pl.when(i == 0)
    def _():
        pT_ref[...] = jnp.transpose(proj_ref[...])
        tab = lut_ref[...]
        mu = tab.mean(axis=1, keepdims=True)
        var = ((tab - mu) ** 2).mean(axis=1, keepdims=True)
        normed = (tab - mu) * lax.rsqrt(var + 1e-6)
        normed = (normed * jnp.reshape(scale_ref[...], (1, FEAT_DIM))
                  + jnp.reshape(bias_ref[...], (1, FEAT_DIM)))
        tabn_ref[...] = jnp.concatenate(
            [normed, jnp.zeros((NUM_BINS, FPAD - FEAT_DIM), jnp.float32)],
            axis=1)

    def kdot(xgT, lo, hi):
        return jnp.dot(jnp.transpose(xgT), pT_ref[pl.ds(lo, hi - lo), :],
                       preferred_element_type=jnp.float32)
    ysr = kdot(st_ref[...], 0, 64)
    ysr = ysr + kdot(at_ref[...], 64, 320)
    ysr = ysr + kdot(bt_ref[...], 320, 832)
    ysr = ysr + kdot(rt_ref[...], 832, 960)
    ys = jnp.transpose(ysr)
    powers = jnp.int32(1) << lax.broadcasted_iota(
        jnp.int32, (HASH_POWER, 1), 0)
    bits = jnp.where(ys > 0, powers, 0)
    hash_ref[...] = bits.sum(axis=0)


def _tc_hash(st, at, bt, rt, proj, lut, scale, bias):
    grid = B // BLK
    return pl.pallas_call(
        _tc_hash_body,
        grid=(grid,),
        in_specs=[
            pl.BlockSpec((64, BLK), lambda i: (0, i)),
            pl.BlockSpec((256, BLK), lambda i: (0, i)),
            pl.BlockSpec((512, BLK), lambda i: (0, i)),
            pl.BlockSpec((128, BLK), lambda i: (0, i)),
            pl.BlockSpec((HASH_POWER, 960), lambda i: (0, 0)),
            pl.BlockSpec((NUM_BINS, FEAT_DIM), lambda i: (0, 0)),
            pl.BlockSpec((FEAT_DIM,), lambda i: (0,)),
            pl.BlockSpec((FEAT_DIM,), lambda i: (0,)),
        ],
        scratch_shapes=[pltpu.VMEM((960, HASH_POWER), jnp.float32)],
        out_specs=[
            pl.BlockSpec((BLK,), lambda i: (i,)),
            pl.BlockSpec((NUM_BINS, FPAD), lambda i: (0, 0)),
        ],
        out_shape=[
            jax.ShapeDtypeStruct((B,), jnp.int32),
            jax.ShapeDtypeStruct((NUM_BINS, FPAD), jnp.float32),
        ],
    )(st, at, bt, rt, proj, lut, scale, bias)


def _make_sc_gather():
    info = plsc.get_sparse_core_info()
    nw = info.num_cores * info.num_subcores
    bpw = B // nw
    ngrp = bpw // 16
    mesh = plsc.VectorSubcoreMesh(core_axis_name="c", subcore_axis_name="s")

    @functools.partial(
        pl.kernel, mesh=mesh,
        compiler_params=pltpu.CompilerParams(needs_layout_passes=False),
        out_type=jax.ShapeDtypeStruct((FEAT_DIM, B), jnp.float32),
        scratch_types=[
            pltpu.VMEM((NUM_BINS, FPAD), jnp.float32),
            pltpu.VMEM((bpw,), jnp.int32),
            pltpu.VMEM((FEAT_DIM, bpw), jnp.float32),
            pltpu.SemaphoreType.DMA,
        ],
    )
    def sc_gather(table_hbm, idx_hbm, out_hbm, tab_v, idx_v, outT_v, sem):
        wid = lax.axis_index("s") * info.num_cores + lax.axis_index("c")
        base = wid * bpw
        cp = pltpu.async_copy(table_hbm, tab_v, sem)
        pltpu.sync_copy(idx_hbm.at[pl.ds(base, bpw)], idx_v)
        cp.wait()
        for g in range(ngrp):
            hvec = idx_v[pl.ds(g * 16, 16)]
            for f in range(FEAT_DIM):
                fvec = jnp.full((16,), f, jnp.int32)
                v = plsc.load_gather(tab_v, [hvec, fvec])
                outT_v[f, pl.ds(g * 16, 16)] = v
        pltpu.sync_copy(outT_v, out_hbm.at[:, pl.ds(base, bpw)])

    return sc_gather


def kernel(self_ob, agents_ob, boxes_ob, ramps_ob, proj_mat, lookup,
           ln_scale, ln_bias, train):
    st = self_ob.T
    at = agents_ob.reshape(B, 256).T
    bt = boxes_ob.reshape(B, 512).T
    rt = ramps_ob.reshape(B, 128).T
    hash_val, table_n = _tc_hash(st, at, bt, rt, proj_mat, lookup,
                                 ln_scale, ln_bias)

    sc_gather = _make_sc_gather()
    outT = sc_gather(table_n, hash_val)
    return outT.T


# speedup vs baseline: 1.1939x; 1.1913x over previous
import functools

import jax
import jax.numpy as jnp
from jax import lax
from jax.experimental import pallas as pl
from jax.experimental.pallas import tpu as pltpu
from jax.experimental.pallas import tpu_sc as plsc

B = 16384
HASH_POWER = 8
NUM_BINS = 256
FEAT_DIM = 32
FPAD = 128
BLK = 2048


def _tc_hash_body(st_ref, at_ref, bt_ref, rt_ref, proj_ref, lut_ref,
                  scale_ref, bias_ref, hash_ref, tabn_ref, pT_ref):
    i = pl.program_id(0)

    @pl.when(i == 0)
    def _():
        pT_ref[...] = jnp.transpose(proj_ref[...])
        tab = lut_ref[...]
        mu = tab.mean(axis=1, keepdims=True)
        var = ((tab - mu) ** 2).mean(axis=1, keepdims=True)
        normed = (tab - mu) * lax.rsqrt(var + 1e-6)
        normed = (normed * jnp.reshape(scale_ref[...], (1, FEAT_DIM))
                  + jnp.reshape(bias_ref[...], (1, FEAT_DIM)))
        tabn_ref[...] = jnp.transpose(normed)

    def kdot(xgT, lo, hi):
        return jnp.dot(jnp.transpose(xgT), pT_ref[pl.ds(lo, hi - lo), :],
                       preferred_element_type=jnp.float32)
    ysr = kdot(st_ref[...], 0, 64)
    ysr = ysr + kdot(at_ref[...], 64, 320)
    ysr = ysr + kdot(bt_ref[...], 320, 832)
    ysr = ysr + kdot(rt_ref[...], 832, 960)
    ys = jnp.transpose(ysr)
    powers = jnp.int32(1) << lax.broadcasted_iota(
        jnp.int32, (HASH_POWER, 1), 0)
    bits = jnp.where(ys > 0, powers, 0)
    hash_ref[...] = bits.sum(axis=0)


def _tc_hash(st, at, bt, rt, proj, lut, scale, bias):
    grid = B // BLK
    return pl.pallas_call(
        _tc_hash_body,
        grid=(grid,),
        in_specs=[
            pl.BlockSpec((64, BLK), lambda i: (0, i)),
            pl.BlockSpec((256, BLK), lambda i: (0, i)),
            pl.BlockSpec((512, BLK), lambda i: (0, i)),
            pl.BlockSpec((128, BLK), lambda i: (0, i)),
            pl.BlockSpec((HASH_POWER, 960), lambda i: (0, 0)),
            pl.BlockSpec((NUM_BINS, FEAT_DIM), lambda i: (0, 0)),
            pl.BlockSpec((FEAT_DIM,), lambda i: (0,)),
            pl.BlockSpec((FEAT_DIM,), lambda i: (0,)),
        ],
        scratch_shapes=[pltpu.VMEM((960, HASH_POWER), jnp.float32)],
        out_specs=[
            pl.BlockSpec((BLK,), lambda i: (i,)),
            pl.BlockSpec((FEAT_DIM, NUM_BINS), lambda i: (0, 0)),
        ],
        out_shape=[
            jax.ShapeDtypeStruct((B,), jnp.int32),
            jax.ShapeDtypeStruct((FEAT_DIM, NUM_BINS), jnp.float32),
        ],
    )(st, at, bt, rt, proj, lut, scale, bias)


def _make_sc_gather():
    info = plsc.get_sparse_core_info()
    nw = info.num_cores * info.num_subcores
    bpw = B // nw
    ngrp = bpw // 16
    mesh = plsc.VectorSubcoreMesh(core_axis_name="c", subcore_axis_name="s")

    @functools.partial(
        pl.kernel, mesh=mesh,
        compiler_params=pltpu.CompilerParams(needs_layout_passes=False),
        out_type=jax.ShapeDtypeStruct((FEAT_DIM, B), jnp.float32),
        scratch_types=[
            pltpu.VMEM((FEAT_DIM, NUM_BINS), jnp.float32),
            pltpu.VMEM((bpw,), jnp.int32),
            pltpu.VMEM((FEAT_DIM, bpw), jnp.float32),
            pltpu.SemaphoreType.DMA,
        ],
    )
    def sc_gather(table_hbm, idx_hbm, out_hbm, tab_v, idx_v, outT_v, sem):
        wid = lax.axis_index("s") * info.num_cores + lax.axis_index("c")
        base = wid * bpw
        cp = pltpu.async_copy(table_hbm, tab_v, sem)
        pltpu.sync_copy(idx_hbm.at[pl.ds(base, bpw)], idx_v)
        cp.wait()
        for g in range(ngrp):
            hvec = idx_v[pl.ds(g * 16, 16)]
            for f in range(FEAT_DIM):
                fvec = jnp.full((16,), f, jnp.int32)
                v = plsc.load_gather(tab_v, [fvec, hvec])
                outT_v[f, pl.ds(g * 16, 16)] = v
        pltpu.sync_copy(outT_v, out_hbm.at[:, pl.ds(base, bpw)])

    return sc_gather


def kernel(self_ob, agents_ob, boxes_ob, ramps_ob, proj_mat, lookup,
           ln_scale, ln_bias, train):
    st = self_ob.T
    at = agents_ob.reshape(B, 256).T
    bt = boxes_ob.reshape(B, 512).T
    rt = ramps_ob.reshape(B, 128).T
    hash_val, table_n = _tc_hash(st, at, bt, rt, proj_mat, lookup,
                                 ln_scale, ln_bias)

    sc_gather = _make_sc_gather()
    outT = sc_gather(table_n, hash_val)
    return outT.T


# speedup vs baseline: 1.1966x; 1.0023x over previous
import functools

import jax
import jax.numpy as jnp
from jax import lax
from jax.experimental import pallas as pl
from jax.experimental.pallas import tpu as pltpu
from jax.experimental.pallas import tpu_sc as plsc

B = 16384
HASH_POWER = 8
NUM_BINS = 256
FEAT_DIM = 32
FPAD = 128
BLK = 2048


def _tc_hash_body(st_ref, at_ref, bt_ref, rt_ref, proj_ref, lut_ref,
                  scale_ref, bias_ref, hash_ref, tabn_ref, pT_ref):
    i = pl.program_id(0)

    @pl.when(i == 0)
    def _():
        pT_ref[...] = jnp.transpose(proj_ref[...])
        tab = lut_ref[...]
        mu = tab.mean(axis=1, keepdims=True)
        var = ((tab - mu) ** 2).mean(axis=1, keepdims=True)
        normed = (tab - mu) * lax.rsqrt(var + 1e-6)
        normed = (normed * jnp.reshape(scale_ref[...], (1, FEAT_DIM))
                  + jnp.reshape(bias_ref[...], (1, FEAT_DIM)))
        tabn_ref[...] = jnp.transpose(normed)

    def kdot(xgT, lo, hi):
        return jnp.dot(jnp.transpose(xgT), pT_ref[pl.ds(lo, hi - lo), :],
                       preferred_element_type=jnp.float32)
    ysr = kdot(st_ref[...], 0, 64)
    ysr = ysr + kdot(at_ref[...], 64, 320)
    ysr = ysr + kdot(bt_ref[...], 320, 832)
    ysr = ysr + kdot(rt_ref[...], 832, 960)
    ys = jnp.transpose(ysr)
    powers = jnp.int32(1) << lax.broadcasted_iota(
        jnp.int32, (HASH_POWER, 1), 0)
    bits = jnp.where(ys > 0, powers, 0)
    hash_ref[...] = bits.sum(axis=0)


def _tc_hash(st, at, bt, rt, proj, lut, scale, bias):
    grid = B // BLK
    return pl.pallas_call(
        _tc_hash_body,
        grid=(grid,),
        in_specs=[
            pl.BlockSpec((64, BLK), lambda i: (0, i)),
            pl.BlockSpec((256, BLK), lambda i: (0, i)),
            pl.BlockSpec((512, BLK), lambda i: (0, i)),
            pl.BlockSpec((128, BLK), lambda i: (0, i)),
            pl.BlockSpec((HASH_POWER, 960), lambda i: (0, 0)),
            pl.BlockSpec((NUM_BINS, FEAT_DIM), lambda i: (0, 0)),
            pl.BlockSpec((FEAT_DIM,), lambda i: (0,)),
            pl.BlockSpec((FEAT_DIM,), lambda i: (0,)),
        ],
        scratch_shapes=[pltpu.VMEM((960, HASH_POWER), jnp.float32)],
        out_specs=[
            pl.BlockSpec((BLK,), lambda i: (i,)),
            pl.BlockSpec((FEAT_DIM, NUM_BINS), lambda i: (0, 0)),
        ],
        out_shape=[
            jax.ShapeDtypeStruct((B,), jnp.int32),
            jax.ShapeDtypeStruct((FEAT_DIM, NUM_BINS), jnp.float32),
        ],
    )(st, at, bt, rt, proj, lut, scale, bias)


def _make_sc_gather():
    info = plsc.get_sparse_core_info()
    nw = info.num_cores * info.num_subcores
    bpw = B // nw
    ngrp = bpw // 16
    mesh = plsc.VectorSubcoreMesh(core_axis_name="c", subcore_axis_name="s")

    @functools.partial(
        pl.kernel, mesh=mesh,
        compiler_params=pltpu.CompilerParams(needs_layout_passes=False),
        out_type=jax.ShapeDtypeStruct((FEAT_DIM, B), jnp.float32),
        scratch_types=[
            pltpu.VMEM((FEAT_DIM, NUM_BINS), jnp.float32),
            pltpu.VMEM((bpw,), jnp.int32),
            pltpu.VMEM((FEAT_DIM, bpw), jnp.float32),
            pltpu.SemaphoreType.DMA,
        ],
    )
    def sc_gather(table_hbm, idx_hbm, out_hbm, tab_v, idx_v, outT_v, sem):
        wid = lax.axis_index("s") * info.num_cores + lax.axis_index("c")
        base = wid * bpw
        cp = pltpu.async_copy(table_hbm, tab_v, sem)
        pltpu.sync_copy(idx_hbm.at[pl.ds(base, bpw)], idx_v)
        cp.wait()
        writes = []
        nq = 4
        gq = ngrp // nq
        for q in range(nq):
            for g in range(q * gq, (q + 1) * gq):
                hvec = idx_v[pl.ds(g * 16, 16)]
                for f in range(FEAT_DIM):
                    fvec = jnp.full((16,), f, jnp.int32)
                    v = plsc.load_gather(tab_v, [fvec, hvec])
                    outT_v[f, pl.ds(g * 16, 16)] = v
            writes.append(pltpu.async_copy(
                outT_v.at[:, pl.ds(q * gq * 16, gq * 16)],
                out_hbm.at[:, pl.ds(base + q * gq * 16, gq * 16)], sem))
        for w in writes:
            w.wait()

    return sc_gather


def kernel(self_ob, agents_ob, boxes_ob, ramps_ob, proj_mat, lookup,
           ln_scale, ln_bias, train):
    st = self_ob.T
    at = agents_ob.reshape(B, 256).T
    bt = boxes_ob.reshape(B, 512).T
    rt = ramps_ob.reshape(B, 128).T
    hash_val, table_n = _tc_hash(st, at, bt, rt, proj_mat, lookup,
                                 ln_scale, ln_bias)

    sc_gather = _make_sc_gather()
    outT = sc_gather(table_n, hash_val)
    return outT.T


# speedup vs baseline: 1.2007x; 1.0034x over previous
import functools

import jax
import jax.numpy as jnp
from jax import lax
from jax.experimental import pallas as pl
from jax.experimental.pallas import tpu as pltpu
from jax.experimental.pallas import tpu_sc as plsc

B = 16384
HASH_POWER = 8
NUM_BINS = 256
FEAT_DIM = 32
BLK = 2048


def _tc_hash_body(st_ref, at_ref, bt_ref, rt_ref, proj_ref, lut_ref,
                  scale_ref, bias_ref, hash_ref, tabn_ref, pT_ref):
    i = pl.program_id(0)

    @pl.when(i == 0)
    def _():
        pT_ref[...] = jnp.transpose(proj_ref[...])
        tab = lut_ref[...]
        mu = tab.mean(axis=1, keepdims=True)
        var = ((tab - mu) ** 2).mean(axis=1, keepdims=True)
        normed = (tab - mu) * lax.rsqrt(var + 1e-6)
        normed = (normed * jnp.reshape(scale_ref[...], (1, FEAT_DIM))
                  + jnp.reshape(bias_ref[...], (1, FEAT_DIM)))
        tabn_ref[...] = jnp.transpose(normed)

    def kdot(xgT, lo, hi):
        return jnp.dot(jnp.transpose(xgT), pT_ref[pl.ds(lo, hi - lo), :],
                       preferred_element_type=jnp.float32)
    ysr = kdot(st_ref[...], 0, 64)
    ysr = ysr + kdot(at_ref[...], 64, 320)
    ysr = ysr + kdot(bt_ref[...], 320, 832)
    ysr = ysr + kdot(rt_ref[...], 832, 960)
    ys = jnp.transpose(ysr)
    powers = jnp.int32(1) << lax.broadcasted_iota(
        jnp.int32, (HASH_POWER, 1), 0)
    bits = jnp.where(ys > 0, powers, 0)
    hash_ref[...] = bits.sum(axis=0)


def _tc_hash(st, at, bt, rt, proj, lut, scale, bias):
    grid = B // BLK
    return pl.pallas_call(
        _tc_hash_body,
        grid=(grid,),
        in_specs=[
            pl.BlockSpec((64, BLK), lambda i: (0, i)),
            pl.BlockSpec((256, BLK), lambda i: (0, i)),
            pl.BlockSpec((512, BLK), lambda i: (0, i)),
            pl.BlockSpec((128, BLK), lambda i: (0, i)),
            pl.BlockSpec((HASH_POWER, 960), lambda i: (0, 0)),
            pl.BlockSpec((NUM_BINS, FEAT_DIM), lambda i: (0, 0)),
            pl.BlockSpec((FEAT_DIM,), lambda i: (0,)),
            pl.BlockSpec((FEAT_DIM,), lambda i: (0,)),
        ],
        scratch_shapes=[pltpu.VMEM((960, HASH_POWER), jnp.float32)],
        out_specs=[
            pl.BlockSpec((BLK,), lambda i: (i,)),
            pl.BlockSpec((FEAT_DIM, NUM_BINS), lambda i: (0, 0)),
        ],
        out_shape=[
            jax.ShapeDtypeStruct((B,), jnp.int32),
            jax.ShapeDtypeStruct((FEAT_DIM, NUM_BINS), jnp.float32),
        ],
    )(st, at, bt, rt, proj, lut, scale, bias)


def _make_sc_gather():
    info = plsc.get_sparse_core_info()
    nw = info.num_cores * info.num_subcores
    bpw = B // nw
    ngrp = bpw // 16
    mesh = plsc.VectorSubcoreMesh(core_axis_name="c", subcore_axis_name="s")

    @functools.partial(
        pl.kernel, mesh=mesh,
        compiler_params=pltpu.CompilerParams(needs_layout_passes=False),
        out_type=jax.ShapeDtypeStruct((FEAT_DIM, B), jnp.float32),
        scratch_types=[
            pltpu.VMEM((FEAT_DIM, NUM_BINS), jnp.float32),
            pltpu.VMEM((bpw,), jnp.int32),
            pltpu.VMEM((FEAT_DIM, bpw), jnp.float32),
            pltpu.SemaphoreType.DMA,
        ],
    )
    def sc_gather(table_hbm, idx_hbm, out_hbm, tab_v, idx_v, outT_v, sem):
        wid = lax.axis_index("s") * info.num_cores + lax.axis_index("c")
        base = wid * bpw
        cp = pltpu.async_copy(table_hbm, tab_v, sem)
        pltpu.sync_copy(idx_hbm.at[pl.ds(base, bpw)], idx_v)
        cp.wait()
        writes = []
        nq = 4
        gq = ngrp // nq
        for q in range(nq):
            for g in range(q * gq, (q + 1) * gq):
                hvec = idx_v[pl.ds(g * 16, 16)]
                for f in range(FEAT_DIM):
                    fvec = jnp.full((16,), f, jnp.int32)
                    v = plsc.load_gather(tab_v, [fvec, hvec])
                    outT_v[f, pl.ds(g * 16, 16)] = v
            writes.append(pltpu.async_copy(
                outT_v.at[:, pl.ds(q * gq * 16, gq * 16)],
                out_hbm.at[:, pl.ds(base + q * gq * 16, gq * 16)], sem))
        for w in writes:
            w.wait()

    return sc_gather


def kernel(self_ob, agents_ob, boxes_ob, ramps_ob, proj_mat, lookup,
           ln_scale, ln_bias, train):
    st = self_ob.T
    at = agents_ob.reshape(B, 256).T
    bt = boxes_ob.reshape(B, 512).T
    rt = ramps_ob.reshape(B, 128).T
    hash_val, table_n = _tc_hash(st, at, bt, rt, proj_mat, lookup,
                                 ln_scale, ln_bias)

    sc_gather = _make_sc_gather()
    outT = sc_gather(table_n, hash_val)
    return outT.T
